# baseline (speedup 1.0000x reference)
import functools

import jax
import jax.numpy as jnp
from jax.experimental import pallas as pl
from jax.experimental.pallas import tpu as pltpu

B, L, M, D = 4, 2048, 8, 768
NUM_REMAIN = 4
TL = 512


def _body(noise_ref, pm_ref, d0, d1, d2, d3, d4, d5, d6, d7,
          out_data_ref, out_rmask_ref, out_remain_ref, out_masked_ref,
          out_revert_ref):
    n = noise_ref[0]
    data = (d0, d1, d2, d3, d4, d5, d6, d7)

    ranks = []
    for m in range(M):
        nm = n[:, m:m + 1]
        acc = jnp.zeros((TL, 1), dtype=jnp.int32)
        for mp in range(M):
            if mp == m:
                continue
            nmp = n[:, mp:mp + 1]
            lt = nmp < nm
            if mp < m:
                lt = jnp.logical_or(lt, nmp == nm)
            acc = acc + lt.astype(jnp.int32)
        ranks.append(acc)

    out_revert_ref[0] = jnp.concatenate(ranks, axis=1)

    for r in range(NUM_REMAIN):
        rem = jnp.zeros((TL, 1), dtype=jnp.int32)
        msk = jnp.zeros((TL, 1), dtype=jnp.int32)
        for m in range(M):
            mi = jnp.int32(m)
            rem = rem + jnp.where(ranks[m] == r, mi, 0)
            msk = msk + jnp.where(ranks[m] == r + NUM_REMAIN, mi, 0)
        out_remain_ref[0, :, r:r + 1] = rem
        out_masked_ref[0, :, r:r + 1] = msk

    pmv = pm_ref[0]
    out_rmask_ref[0] = jnp.broadcast_to(pmv, (TL, NUM_REMAIN))

    for r in range(NUM_REMAIN):
        acc = jnp.zeros((TL, D), dtype=jnp.float32)
        for m in range(M):
            acc = jnp.where(ranks[m] == r, data[m][0], acc)
        out_data_ref[0, :, r * D:(r + 1) * D] = acc


@functools.partial(jax.jit, static_argnums=())
def _run(noise, pm, data):
    grid = (B, L // TL)
    data_spec = pl.BlockSpec((1, TL, D), lambda b, i: (b, i, 0))
    outs = pl.pallas_call(
        _body,
        grid=grid,
        in_specs=[
            pl.BlockSpec((1, TL, M), lambda b, i: (b, i, 0)),
            pl.BlockSpec((1, TL, 1), lambda b, i: (b, i, 0)),
        ] + [data_spec] * M,
        out_specs=[
            pl.BlockSpec((1, TL, NUM_REMAIN * D), lambda b, i: (b, i, 0)),
            pl.BlockSpec((1, TL, NUM_REMAIN), lambda b, i: (b, i, 0)),
            pl.BlockSpec((1, TL, NUM_REMAIN), lambda b, i: (b, i, 0)),
            pl.BlockSpec((1, TL, NUM_REMAIN), lambda b, i: (b, i, 0)),
            pl.BlockSpec((1, TL, M), lambda b, i: (b, i, 0)),
        ],
        out_shape=[
            jax.ShapeDtypeStruct((B, L, NUM_REMAIN * D), jnp.float32),
            jax.ShapeDtypeStruct((B, L, NUM_REMAIN), jnp.float32),
            jax.ShapeDtypeStruct((B, L, NUM_REMAIN), jnp.int32),
            jax.ShapeDtypeStruct((B, L, NUM_REMAIN), jnp.int32),
            jax.ShapeDtypeStruct((B, L, M), jnp.int32),
        ],
        compiler_params=pltpu.CompilerParams(
            dimension_semantics=("parallel", "parallel"),
        ),
    )(noise, pm, *data)
    return outs


def kernel(data_0, data_1, data_2, data_3, data_4, data_5, data_6, data_7,
           temporal_padding_mask):
    data = (data_0, data_1, data_2, data_3, data_4, data_5, data_6, data_7)
    noise = jax.random.uniform(jax.random.key(42), (B, L, M))
    pm = jnp.concatenate(
        [jnp.ones((B, 1, 1), temporal_padding_mask.dtype), temporal_padding_mask],
        axis=1)
    res = _run(noise, pm, data)
    remained_flat, remain_mask, remain_idx, masked_idx, revert_idx = res
    remained_data = remained_flat.reshape(B, L, NUM_REMAIN, D)
    return (remained_data, remain_mask, remain_idx, masked_idx, revert_idx, pm)


# speedup vs baseline: 1.4396x; 1.4396x over previous
import functools

import jax
import jax.numpy as jnp
from jax.experimental import pallas as pl
from jax.experimental.pallas import tpu as pltpu

B, L, M, D = 4, 2048, 8, 768
NR = 4
TL = 256


def _index_body(noise_ref, pm_ref, rev_ref, rem_ref, msk_ref, rmask_ref,
                bits_ref):
    n = noise_ref[0]

    ranks = []
    for m in range(M):
        nm = n[m:m + 1, :]
        acc = jnp.zeros((1, L), dtype=jnp.int32)
        for mp in range(M):
            if mp == m:
                continue
            nmp = n[mp:mp + 1, :]
            lt = nmp < nm
            if mp < m:
                lt = jnp.logical_or(lt, nmp == nm)
            acc = acc + lt.astype(jnp.int32)
        ranks.append(acc)
        rev_ref[0, m:m + 1, :] = acc

    for r in range(NR):
        rem = jnp.zeros((1, L), dtype=jnp.int32)
        msk = jnp.zeros((1, L), dtype=jnp.int32)
        for m in range(M):
            mi = jnp.int32(m)
            rem = rem + jnp.where(ranks[m] == r, mi, 0)
            msk = msk + jnp.where(ranks[m] == r + NR, mi, 0)
        rem_ref[0, r:r + 1, :] = rem
        msk_ref[0, r:r + 1, :] = msk
        for k in range(3):
            bits_ref[0, r * 3 + k:r * 3 + k + 1, :] = (
                jnp.right_shift(rem, k) & 1)
        rmask_ref[0, r:r + 1, :] = pm_ref[0]


def _data_body(b0r, b1r, b2r, b3r, d0, d1, d2, d3, d4, d5, d6, d7, out_ref):
    data = (d0[0], d1[0], d2[0], d3[0], d4[0], d5[0], d6[0], d7[0])
    bits = (b0r, b1r, b2r, b3r)
    for r in range(NR):
        bb = bits[r][0]
        b0 = bb[:, 0:1] != 0
        b1 = bb[:, 1:2] != 0
        b2 = bb[:, 2:3] != 0
        t0 = jnp.where(b0, data[1], data[0])
        t1 = jnp.where(b0, data[3], data[2])
        t2 = jnp.where(b0, data[5], data[4])
        t3 = jnp.where(b0, data[7], data[6])
        u0 = jnp.where(b1, t1, t0)
        u1 = jnp.where(b1, t3, t2)
        out_ref[0, :, r * D:(r + 1) * D] = jnp.where(b2, u1, u0)


@jax.jit
def _run(noise_t, pm_t, data):
    rev_t, rem_t, msk_t, rmask_t, bits_t = pl.pallas_call(
        _index_body,
        grid=(B,),
        in_specs=[
            pl.BlockSpec((1, M, L), lambda b: (b, 0, 0)),
            pl.BlockSpec((1, 1, L), lambda b: (b, 0, 0)),
        ],
        out_specs=[
            pl.BlockSpec((1, M, L), lambda b: (b, 0, 0)),
            pl.BlockSpec((1, NR, L), lambda b: (b, 0, 0)),
            pl.BlockSpec((1, NR, L), lambda b: (b, 0, 0)),
            pl.BlockSpec((1, NR, L), lambda b: (b, 0, 0)),
            pl.BlockSpec((1, 3 * NR, L), lambda b: (b, 0, 0)),
        ],
        out_shape=[
            jax.ShapeDtypeStruct((B, M, L), jnp.int32),
            jax.ShapeDtypeStruct((B, NR, L), jnp.int32),
            jax.ShapeDtypeStruct((B, NR, L), jnp.int32),
            jax.ShapeDtypeStruct((B, NR, L), jnp.float32),
            jax.ShapeDtypeStruct((B, 3 * NR, L), jnp.int32),
        ],
        compiler_params=pltpu.CompilerParams(
            dimension_semantics=("parallel",),
        ),
    )(noise_t, pm_t)

    bits = [bits_t[:, r * 3:(r + 1) * 3, :].transpose(0, 2, 1) for r in range(NR)]

    data_spec = pl.BlockSpec((1, TL, D), lambda b, i: (b, i, 0))
    bits_spec = pl.BlockSpec((1, TL, 3), lambda b, i: (b, i, 0))
    remained = pl.pallas_call(
        _data_body,
        grid=(B, L // TL),
        in_specs=[bits_spec] * NR + [data_spec] * M,
        out_specs=pl.BlockSpec((1, TL, NR * D), lambda b, i: (b, i, 0)),
        out_shape=jax.ShapeDtypeStruct((B, L, NR * D), jnp.float32),
        compiler_params=pltpu.CompilerParams(
            dimension_semantics=("parallel", "parallel"),
        ),
    )(*bits, *data)
    return remained, rev_t, rem_t, msk_t, rmask_t


def kernel(data_0, data_1, data_2, data_3, data_4, data_5, data_6, data_7,
           temporal_padding_mask):
    data = (data_0, data_1, data_2, data_3, data_4, data_5, data_6, data_7)
    noise_t = jax.random.uniform(jax.random.key(42), (B, L, M)).transpose(0, 2, 1)
    pm = jnp.concatenate(
        [jnp.ones((B, 1, 1), temporal_padding_mask.dtype), temporal_padding_mask],
        axis=1)
    pm_t = pm.transpose(0, 2, 1)
    remained, rev_t, rem_t, msk_t, rmask_t = _run(noise_t, pm_t, data)
    remained_data = remained.reshape(B, L, NR, D)
    remain_idx = rem_t.transpose(0, 2, 1)
    masked_idx = msk_t.transpose(0, 2, 1)
    revert_idx = rev_t.transpose(0, 2, 1)
    remain_padding_mask = rmask_t.transpose(0, 2, 1)
    return (remained_data, remain_padding_mask, remain_idx, masked_idx,
            revert_idx, pm)
